# baseline (speedup 1.0000x reference)
import jax
import jax.numpy as jnp
from jax import lax
from jax.experimental import pallas as pl
from jax.experimental.pallas import tpu as pltpu
from jax.experimental.pallas import tpu_sc as plsc

NUM_EMBEDDINGS = 1000000
EMBED_DIM = 64
EPS = 1e-5

NC = 2
NS = 16
NW = NC * NS

B = 4096 * 200
PER_W = B // NW
CHUNK = 256
N_CHUNKS = PER_W // CHUNK
IDX_ROWS = CHUNK // 128

BLK_B = 32


def _gather_body(m_hbm, table_hbm, out_hbm, idx_v, rows_v, gsem0, gsem1):
    wid = lax.axis_index("s") * NC + lax.axis_index("c")
    idx_row0 = wid * (PER_W // 128)
    out_row0 = wid * PER_W
    gsems = (gsem0, gsem1)

    def load_idx(ci, b):
        pltpu.sync_copy(
            m_hbm.at[pl.ds(idx_row0 + ci * IDX_ROWS, IDX_ROWS)], idx_v.at[b])

    def fire(b):
        for j in range(IDX_ROWS):
            pltpu.async_copy(table_hbm.at[idx_v.at[b, j]],
                             rows_v.at[b, pl.ds(j * 128, 128)], gsems[b])

    def wait_gathers(b):
        for j in range(IDX_ROWS):
            pltpu.make_async_copy(table_hbm.at[idx_v.at[b, j]],
                                  rows_v.at[b, pl.ds(j * 128, 128)],
                                  gsems[b]).wait()

    def copy_out(ci, b):
        pltpu.sync_copy(rows_v.at[b],
                        out_hbm.at[pl.ds(out_row0 + ci * CHUNK, CHUNK)])

    def step(ci, b):
        nb = 1 - b
        load_idx(ci + 1, nb)
        fire(nb)
        wait_gathers(b)
        copy_out(ci, b)

    load_idx(0, 0)
    fire(0)

    def pair_body(k, carry):
        step(2 * k, 0)
        step(2 * k + 1, 1)
        return carry

    lax.fori_loop(0, N_CHUNKS // 2 - 1, pair_body, 0)
    step(N_CHUNKS - 2, 0)
    wait_gathers(1)
    copy_out(N_CHUNKS - 1, 1)


def _sc_gather(m2, t2):
    mesh = plsc.VectorSubcoreMesh(core_axis_name="c", subcore_axis_name="s",
                                  num_cores=NC, num_subcores=NS)
    return pl.kernel(
        _gather_body,
        out_type=jax.ShapeDtypeStruct((B, 128), jnp.float32),
        mesh=mesh,
        compiler_params=pltpu.CompilerParams(needs_layout_passes=False),
        scratch_types=[
            pltpu.VMEM((2, IDX_ROWS, 128), jnp.int32),
            pltpu.VMEM((2, CHUNK, 128), jnp.float32),
            pltpu.SemaphoreType.DMA,
            pltpu.SemaphoreType.DMA,
        ],
    )(m2, t2)


def _ln_body(g_ref, x_ref, gamma_ref, beta_ref, out_ref):
    xg = g_ref[...]
    par = x_ref[...] & 1
    lo = xg[:, :EMBED_DIM]
    hi = xg[:, EMBED_DIM:]
    g = gamma_ref[0, :]
    b = beta_ref[0, :]

    def norm(x):
        mean = jnp.mean(x, axis=1, keepdims=True)
        xc = x - mean
        var = jnp.mean(xc * xc, axis=1, keepdims=True)
        o = xc * lax.rsqrt(var + EPS) * g + b
        return o.reshape(BLK_B, 200, EMBED_DIM)

    par3 = lax.broadcast_in_dim(par, (BLK_B, 200, EMBED_DIM), (0, 1))
    out_ref[...] = jnp.where(par3 == 1, norm(hi), norm(lo))


def _tc_layernorm(g2, x, gamma2, beta2):
    return pl.pallas_call(
        _ln_body,
        grid=(4096 // BLK_B,),
        in_specs=[
            pl.BlockSpec((BLK_B * 200, 128), lambda i: (i, 0)),
            pl.BlockSpec((BLK_B, 200), lambda i: (i, 0)),
            pl.BlockSpec((1, EMBED_DIM), lambda i: (0, 0)),
            pl.BlockSpec((1, EMBED_DIM), lambda i: (0, 0)),
        ],
        out_specs=pl.BlockSpec((BLK_B, 200, EMBED_DIM), lambda i: (i, 0, 0)),
        out_shape=jax.ShapeDtypeStruct((4096, 200, EMBED_DIM), jnp.float32),
    )(g2, x, gamma2, beta2)


@jax.jit
def kernel(x, table, gamma, beta):
    xi = x.astype(jnp.int32)
    m2 = (xi >> 1).reshape(B // 128, 128)
    t2 = table.reshape(NUM_EMBEDDINGS // 2, 2 * EMBED_DIM)
    g2 = _sc_gather(m2, t2)
    return _tc_layernorm(g2, xi, gamma.reshape(1, EMBED_DIM),
                         beta.reshape(1, EMBED_DIM))


# speedup vs baseline: 1.5814x; 1.5814x over previous
import jax
import jax.numpy as jnp
from jax import lax
from jax.experimental import pallas as pl
from jax.experimental.pallas import tpu as pltpu
from jax.experimental.pallas import tpu_sc as plsc

NUM_EMBEDDINGS = 1000000
EMBED_DIM = 64
EPS = 1e-5

NC = 2
NS = 16
NW = NC * NS

B = 4096 * 200
PER_W = B // NW
CHUNK = 512
N_CHUNKS = PER_W // CHUNK
IDX_ROWS = CHUNK // 128

BLK_B = 32
RB = BLK_B * 200 // 2


def _gather_body(x_hbm, table_hbm, out_hbm, idx_v, rows_v, gsem0, gsem1):
    wid = lax.axis_index("s") * NC + lax.axis_index("c")
    idx_row0 = wid * (PER_W // 128)
    out_row0 = wid * PER_W
    gsems = (gsem0, gsem1)

    def load_idx(ci, b):
        pltpu.sync_copy(
            x_hbm.at[pl.ds(idx_row0 + ci * IDX_ROWS, IDX_ROWS)], idx_v.at[b])

    def fire(b):
        for j in range(IDX_ROWS):
            pltpu.async_copy(table_hbm.at[idx_v.at[b, j]],
                             rows_v.at[b, pl.ds(j * 128, 128)], gsems[b])

    def wait_gathers(b):
        for j in range(IDX_ROWS):
            pltpu.make_async_copy(table_hbm.at[idx_v.at[b, j]],
                                  rows_v.at[b, pl.ds(j * 128, 128)],
                                  gsems[b]).wait()

    def copy_out(ci, b):
        pltpu.sync_copy(rows_v.at[b],
                        out_hbm.at[pl.ds(out_row0 + ci * CHUNK, CHUNK)])

    def step(ci, b):
        nb = 1 - b
        load_idx(ci + 1, nb)
        fire(nb)
        wait_gathers(b)
        copy_out(ci, b)

    load_idx(0, 0)
    fire(0)

    def pair_body(k, carry):
        step(2 * k, 0)
        step(2 * k + 1, 1)
        return carry

    lax.fori_loop(0, N_CHUNKS // 2 - 1, pair_body, 0)
    step(N_CHUNKS - 2, 0)
    wait_gathers(1)
    copy_out(N_CHUNKS - 1, 1)


def _sc_gather(xf, table):
    mesh = plsc.VectorSubcoreMesh(core_axis_name="c", subcore_axis_name="s",
                                  num_cores=NC, num_subcores=NS)
    return pl.kernel(
        _gather_body,
        out_type=jax.ShapeDtypeStruct((B, EMBED_DIM), jnp.float32),
        mesh=mesh,
        compiler_params=pltpu.CompilerParams(needs_layout_passes=False,
                                             use_tc_tiling_on_sc=False),
        scratch_types=[
            pltpu.VMEM((2, IDX_ROWS, 128), jnp.int32),
            pltpu.VMEM((2, CHUNK, EMBED_DIM), jnp.float32),
            pltpu.SemaphoreType.DMA,
            pltpu.SemaphoreType.DMA,
        ],
    )(xf, table)


def _ln_body(g_ref, gamma_ref, beta_ref, out_ref):
    xg = g_ref[...]
    g = gamma_ref[0, :]
    b = beta_ref[0, :]

    def norm(x):
        mean = jnp.mean(x, axis=1, keepdims=True)
        xc = x - mean
        var = jnp.mean(xc * xc, axis=1, keepdims=True)
        return xc * lax.rsqrt(var + EPS) * g + b

    o = jnp.concatenate(
        [norm(xg[:, :EMBED_DIM]), norm(xg[:, EMBED_DIM:])], axis=1)
    out_ref[...] = o.reshape(BLK_B, 100, 128)


def _tc_layernorm(g2, gamma2, beta2):
    return pl.pallas_call(
        _ln_body,
        grid=(4096 // BLK_B,),
        in_specs=[
            pl.BlockSpec((RB, 128), lambda i: (i, 0)),
            pl.BlockSpec((1, EMBED_DIM), lambda i: (0, 0)),
            pl.BlockSpec((1, EMBED_DIM), lambda i: (0, 0)),
        ],
        out_specs=pl.BlockSpec((BLK_B, 100, 128), lambda i: (i, 0, 0)),
        out_shape=jax.ShapeDtypeStruct((4096, 100, 128), jnp.float32),
    )(g2, gamma2, beta2)


@jax.jit
def kernel(x, table, gamma, beta):
    xf = x.astype(jnp.int32).reshape(B // 128, 128)
    g2 = _sc_gather(xf, table).reshape(B // 2, 128)
    out = _tc_layernorm(g2, gamma.reshape(1, EMBED_DIM),
                        beta.reshape(1, EMBED_DIM))
    return out.reshape(4096, 200, EMBED_DIM)
